# baseline (speedup 1.0000x reference)
import functools

import jax
import jax.numpy as jnp
import numpy as np
from jax import lax
from jax.experimental import pallas as pl
from jax.experimental.pallas import tpu as pltpu
from jax.experimental.pallas import tpu_sc as plsc


def _sc_pool(x1, embp, e, nc, ns, seq, gb):
    nw = nc * ns
    rpw = x1.shape[0] // (nw * seq)
    gseq = gb * seq
    ngather = rpw // gb
    ep = embp.shape[1]
    nlane = 16
    nv = ep // nlane
    inv_l = jnp.float32(1.0 / seq)
    mask_hi = jnp.int32(-65536)

    mesh = plsc.VectorSubcoreMesh(core_axis_name="c", subcore_axis_name="s")
    nbuf = 4
    chunk = 64
    nchunk = rpw // chunk
    gpc = chunk // gb
    assert gpc % nbuf == 0 and nchunk % 2 == 0

    @functools.partial(
        pl.kernel,
        mesh=mesh,
        compiler_params=pltpu.CompilerParams(use_tc_tiling_on_sc=False),
        out_type=jax.ShapeDtypeStruct((nw * rpw, e), jnp.float32),
        scratch_types=[
            pltpu.VMEM((rpw * seq,), jnp.int32),
            [pltpu.VMEM((gseq, ep), jnp.float32) for _ in range(nbuf)],
            [pltpu.VMEM((chunk, e), jnp.float32) for _ in range(2)],
            [pltpu.SemaphoreType.DMA for _ in range(nbuf)],
            [pltpu.SemaphoreType.DMA for _ in range(2)],
        ],
    )
    def body(x_hbm, emb_hbm, out_hbm, idx_v, rows_bufs, pool_bufs,
             gsems, osems):
        wid = lax.axis_index("s") * nc + lax.axis_index("c")
        base = wid * rpw
        pltpu.sync_copy(x_hbm.at[pl.ds(wid * rpw * seq, rpw * seq)], idx_v)

        def start(g, buf, sem):
            idx = idx_v.at[pl.ds(pl.multiple_of(g * gseq, 8), gseq)]
            pltpu.make_async_copy(emb_hbm.at[idx], buf, sem).start()

        def wait(buf, sem):
            idx = idx_v.at[pl.ds(0, gseq)]
            pltpu.make_async_copy(emb_hbm.at[idx], buf, sem).wait()

        def accum(buf, row0, pool_buf, lr):
            def inner(j, acc):
                out = []
                for k in range(nv):
                    w = buf[row0 + j, pl.ds(nlane * k, nlane)]
                    wi = lax.bitcast_convert_type(w, jnp.int32)
                    lo = lax.bitcast_convert_type(wi << 16, jnp.float32)
                    hi = lax.bitcast_convert_type(wi & mask_hi, jnp.float32)
                    out.append(acc[2 * k] + lo)
                    out.append(acc[2 * k + 1] + hi)
                return tuple(out)

            acc = lax.fori_loop(
                0, seq, inner,
                tuple(jnp.zeros((nlane,), jnp.float32) for _ in range(2 * nv)),
                unroll=2)
            for k in range(nv):
                pool_buf[lr, pl.ds(2 * nlane * k, nlane)] = acc[2 * k] * inv_l
                pool_buf[lr, pl.ds(2 * nlane * k + nlane, nlane)] = (
                    acc[2 * k + 1] * inv_l)

        for bb in range(nbuf):
            start(bb, rows_bufs[bb], gsems[bb])

        def outer(c2, _):
            for cc in range(2):
                c = 2 * c2 + cc
                pool_buf, osem = pool_bufs[cc], osems[cc]
                out_slc = out_hbm.at[pl.ds(base + c * chunk, chunk)]

                @pl.when(c2 > 0)
                def _():
                    pltpu.make_async_copy(pool_buf, out_slc, osem).wait()

                def ring(q, _):
                    for bb in range(nbuf):
                        lg = q * nbuf + bb
                        g = c * gpc + lg
                        buf, sem = rows_bufs[bb], gsems[bb]
                        wait(buf, sem)

                        @pl.when(g + nbuf < ngather)
                        def _():
                            start(g + nbuf, buf, sem)

                        for rb in range(gb):
                            accum(buf, rb * seq, pool_buf, lg * gb + rb)
                    return 0

                lax.fori_loop(0, gpc // nbuf, ring, 0)
                pltpu.make_async_copy(pool_buf, out_slc, osem).start()
            return 0

        lax.fori_loop(0, nchunk // 2, outer, 0)
        for cc in range(2):
            c = nchunk - 2 + cc
            pltpu.make_async_copy(
                pool_bufs[cc],
                out_hbm.at[pl.ds(base + c * chunk, chunk)],
                osems[cc]).wait()

    return body(x1, embp)


def _tc_mlp(pooled, w1, b1, w2, b2, w3, b3, bt):
    b, e = pooled.shape
    h1 = w1.shape[1]
    h2 = w2.shape[1]
    out = w3.shape[1]

    def body(p_ref, w1_ref, b1_ref, w2_ref, b2_ref, w3_ref, b3_ref, o_ref):
        h = jnp.dot(p_ref[...], w1_ref[...], preferred_element_type=jnp.float32)
        h = jnp.maximum(h + b1_ref[...], 0.0)
        h = jnp.dot(h, w2_ref[...], preferred_element_type=jnp.float32)
        h = jnp.maximum(h + b2_ref[...], 0.0)
        h = jnp.dot(h, w3_ref[...], preferred_element_type=jnp.float32)
        o_ref[...] = h + b3_ref[...]

    zero = lambda i: (0, 0)
    return pl.pallas_call(
        body,
        grid=(b // bt,),
        in_specs=[
            pl.BlockSpec((bt, e), lambda i: (i, 0)),
            pl.BlockSpec((e, h1), zero),
            pl.BlockSpec((1, h1), zero),
            pl.BlockSpec((h1, h2), zero),
            pl.BlockSpec((1, h2), zero),
            pl.BlockSpec((h2, out), zero),
            pl.BlockSpec((1, out), zero),
        ],
        out_specs=pl.BlockSpec((bt, out), lambda i: (i, 0)),
        out_shape=jax.ShapeDtypeStruct((b, out), jnp.float32),
    )(pooled, w1, b1.reshape(1, -1), w2, b2.reshape(1, -1),
      w3, b3.reshape(1, -1))


def kernel(x, emb, W1, b1, W2, b2, W3, b3):
    b, seq = x.shape
    v, e = emb.shape
    nc, ns = 2, 16
    nw = nc * ns
    gb = 4
    x1 = x.astype(jnp.int32).reshape(-1)
    emb16 = jax.lax.bitcast_convert_type(emb.astype(jnp.bfloat16), jnp.uint16)
    embp = jax.lax.bitcast_convert_type(
        emb16.reshape(v // 2, 2, e // 2, 2), jnp.float32).reshape(v // 2, e)
    pooled = _sc_pool(x1, embp.reshape(v, e // 2), e, nc, ns, seq, gb)
    perm = np.concatenate(
        [np.concatenate([g * 32 + 2 * np.arange(16),
                         g * 32 + 2 * np.arange(16) + 1])
         for g in range(e // 32)]).astype(np.int32)
    return _tc_mlp(pooled, W1[perm, :], b1, W2, b2, W3, b3, bt=1024)


# speedup vs baseline: 26.4849x; 26.4849x over previous
import functools

import jax
import jax.numpy as jnp
import numpy as np
from jax import lax
from jax.experimental import pallas as pl
from jax.experimental.pallas import tpu as pltpu
from jax.experimental.pallas import tpu_sc as plsc


def _sc_pool(x3, emb, nc, ns, seq):
    nw = nc * ns
    _, ngather, gseq = x3.shape
    gb = gseq // seq
    rpw = ngather * gb
    e = emb.shape[1]
    nlane = 16
    nv = e // nlane
    inv_l = jnp.float32(1.0 / seq)

    mesh = plsc.VectorSubcoreMesh(core_axis_name="c", subcore_axis_name="s")
    nbuf = 4
    chunk = 64
    nchunk = rpw // chunk
    gpc = chunk // gb
    assert gpc % nbuf == 0 and nchunk % 2 == 0

    @functools.partial(
        pl.kernel,
        mesh=mesh,
        out_type=jax.ShapeDtypeStruct((nw * rpw, e), jnp.float32),
        scratch_types=[
            pltpu.VMEM((ngather, gseq), jnp.int32),
            [pltpu.VMEM((gseq, e), jnp.float32) for _ in range(nbuf)],
            [pltpu.VMEM((chunk, e), jnp.float32) for _ in range(2)],
            [pltpu.SemaphoreType.DMA for _ in range(nbuf)],
            [pltpu.SemaphoreType.DMA for _ in range(2)],
        ],
    )
    def body(x_hbm, emb_hbm, out_hbm, idx_v, rows_bufs, pool_bufs,
             gsems, osems):
        wid = lax.axis_index("s") * nc + lax.axis_index("c")
        base = wid * rpw
        pltpu.sync_copy(x_hbm.at[wid], idx_v)

        def start(g, buf, sem):
            pltpu.make_async_copy(emb_hbm.at[idx_v.at[g]], buf, sem).start()

        def wait(buf, sem):
            pltpu.make_async_copy(emb_hbm.at[idx_v.at[0]], buf, sem).wait()

        def accum(buf, row0, pool_buf, lr):
            def inner(j, acc):
                out = []
                for k in range(nv):
                    w = buf[row0 + j, pl.ds(nlane * k, nlane)]
                    out.append(acc[k] + w)
                return tuple(out)

            acc = lax.fori_loop(
                0, seq, inner,
                tuple(jnp.zeros((nlane,), jnp.float32) for _ in range(nv)),
                unroll=10)
            for k in range(nv):
                pool_buf[lr, pl.ds(nlane * k, nlane)] = acc[k] * inv_l

        for bb in range(nbuf):
            start(bb, rows_bufs[bb], gsems[bb])

        def outer(c2, _):
            for cc in range(2):
                c = 2 * c2 + cc
                pool_buf, osem = pool_bufs[cc], osems[cc]
                out_slc = out_hbm.at[pl.ds(base + c * chunk, chunk)]

                @pl.when(c2 > 0)
                def _():
                    pltpu.make_async_copy(pool_buf, out_slc, osem).wait()

                def ring(q, _):
                    for bb in range(nbuf):
                        lg = q * nbuf + bb
                        g = c * gpc + lg
                        buf, sem = rows_bufs[bb], gsems[bb]
                        wait(buf, sem)

                        for rb in range(gb):
                            accum(buf, rb * seq, pool_buf, lg * gb + rb)

                        @pl.when(g + nbuf < ngather)
                        def _():
                            start(g + nbuf, buf, sem)
                    return 0

                lax.fori_loop(0, gpc // nbuf, ring, 0)
                pltpu.make_async_copy(pool_buf, out_slc, osem).start()
            return 0

        lax.fori_loop(0, nchunk // 2, outer, 0)
        for cc in range(2):
            c = nchunk - 2 + cc
            pltpu.make_async_copy(
                pool_bufs[cc],
                out_hbm.at[pl.ds(base + c * chunk, chunk)],
                osems[cc]).wait()

    return body(x3, emb)


def _tc_mlp(pooled, w1, b1, w2, b2, w3, b3, bt):
    b, e = pooled.shape
    h1 = w1.shape[1]
    h2 = w2.shape[1]
    out = w3.shape[1]

    def body(p_ref, w1_ref, b1_ref, w2_ref, b2_ref, w3_ref, b3_ref, o_ref):
        h = jnp.dot(p_ref[...], w1_ref[...], preferred_element_type=jnp.float32)
        h = jnp.maximum(h + b1_ref[...], 0.0)
        h = jnp.dot(h, w2_ref[...], preferred_element_type=jnp.float32)
        h = jnp.maximum(h + b2_ref[...], 0.0)
        h = jnp.dot(h, w3_ref[...], preferred_element_type=jnp.float32)
        o_ref[...] = h + b3_ref[...]

    zero = lambda i: (0, 0)
    return pl.pallas_call(
        body,
        grid=(b // bt,),
        in_specs=[
            pl.BlockSpec((bt, e), lambda i: (i, 0)),
            pl.BlockSpec((e, h1), zero),
            pl.BlockSpec((1, h1), zero),
            pl.BlockSpec((h1, h2), zero),
            pl.BlockSpec((1, h2), zero),
            pl.BlockSpec((h2, out), zero),
            pl.BlockSpec((1, out), zero),
        ],
        out_specs=pl.BlockSpec((bt, out), lambda i: (i, 0)),
        out_shape=jax.ShapeDtypeStruct((b, out), jnp.float32),
    )(pooled, w1, b1.reshape(1, -1), w2, b2.reshape(1, -1),
      w3, b3.reshape(1, -1))


def kernel(x, emb, W1, b1, W2, b2, W3, b3):
    b, seq = x.shape
    nc, ns = 2, 16
    nw = nc * ns
    rpw = b // nw
    gb = 2
    x3 = x.astype(jnp.int32).reshape(nw, rpw // gb, gb * seq)
    pooled = _sc_pool(x3, emb, nc, ns, seq)
    return _tc_mlp(pooled, W1, b1, W2, b2, W3, b3, bt=1024)
